# baseline (speedup 1.0000x reference)
import jax
import jax.numpy as jnp
from jax import lax
from jax.experimental import pallas as pl
from jax.experimental.pallas import tpu as pltpu
from jax.experimental.pallas import tpu_sc as plsc

_NUM_CLASSES = 100000
_FEAT_DIM = 64
_BATCH = 16384

_NC = 2
_NS = 16
_NW = _NC * _NS
_BPW = _BATCH // _NW
_ICHUNK = 128
_NCHUNK = _BPW // _ICHUNK
_L = 16
_GROUPS = _BPW // _L


def _center_loss_body(labels_hbm, feat_hbm, centers_hbm, out_hbm,
                      labels_v, tidx_v, rows_v, feat_v, acc_v, fsem, gsem):
    wid = lax.axis_index("s") * _NC + lax.axis_index("c")
    base = wid * _BPW

    fcopy = pltpu.async_copy(
        feat_hbm.at[pl.ds(wid * (_BPW // 2), _BPW // 2), :], feat_v, fsem)

    pltpu.sync_copy(labels_hbm.at[pl.ds(base, _BPW)], labels_v)
    for j in range(_NCHUNK):
        for g in range(_ICHUNK // _L):
            lv = labels_v[pl.ds(j * _ICHUNK + g * _L, _L)]
            tidx_v[j, pl.ds(g * _L, _L)] = lax.shift_right_logical(lv, 1)

    gathers = []
    for j in range(_NCHUNK):
        gathers.append(
            pltpu.async_copy(
                centers_hbm.at[tidx_v.at[j]],
                rows_v.at[pl.ds(j * _ICHUNK, _ICHUNK), :],
                gsem,
            )
        )
    for g in gathers:
        g.wait()
    fcopy.wait()

    lane = lax.iota(jnp.int32, _L)
    frow_half = lane // 2
    fcol_half = (lane % 2) * _FEAT_DIM

    acc = jnp.zeros((_L,), jnp.float32)
    for g in range(_GROUPS):
        crow = lane + g * _L
        frow = frow_half + g * (_L // 2)
        pv = labels_v[pl.ds(g * _L, _L)] & 1
        ccol_base = pv * _FEAT_DIM

        def col_body(c0, acc):
            c0v = jnp.full((_L,), 0, jnp.int32) + c0
            cv = plsc.load_gather(rows_v, [crow, ccol_base + c0v])
            fv = plsc.load_gather(feat_v, [frow, fcol_half + c0v])
            d = fv - cv
            return acc + d * d

        acc = lax.fori_loop(0, _FEAT_DIM, col_body, acc, unroll=4)

    acc_v[...] = acc
    pltpu.sync_copy(acc_v, out_hbm.at[wid])


@jax.jit
def _center_loss_sc(labels, features, centers):
    c128 = centers.reshape(_NUM_CLASSES // 2, 2 * _FEAT_DIM)
    f128 = features.reshape(_BATCH // 2, 2 * _FEAT_DIM)
    mesh = plsc.VectorSubcoreMesh(core_axis_name="c", subcore_axis_name="s")
    partials = pl.kernel(
        _center_loss_body,
        mesh=mesh,
        compiler_params=pltpu.CompilerParams(
            use_tc_tiling_on_sc=False, needs_layout_passes=False),
        out_type=jax.ShapeDtypeStruct((_NW, _L), jnp.float32),
        scratch_types=[
            pltpu.VMEM((_BPW,), jnp.int32),
            pltpu.VMEM((_NCHUNK, _ICHUNK), jnp.int32),
            pltpu.VMEM((_BPW, 2 * _FEAT_DIM), jnp.float32),
            pltpu.VMEM((_BPW // 2, 2 * _FEAT_DIM), jnp.float32),
            pltpu.VMEM((_L,), jnp.float32),
            pltpu.SemaphoreType.DMA,
            pltpu.SemaphoreType.DMA,
        ],
    )(labels, f128, c128)
    return jnp.sum(partials) * (0.5 / _BATCH)


def kernel(features, labels, centers):
    return _center_loss_sc(labels.astype(jnp.int32), features, centers)


# speedup vs baseline: 2.8804x; 2.8804x over previous
import jax
import jax.numpy as jnp
from jax import lax
from jax.experimental import pallas as pl
from jax.experimental.pallas import tpu as pltpu
from jax.experimental.pallas import tpu_sc as plsc

_NUM_CLASSES = 100000
_FEAT_DIM = 64
_BATCH = 16384

_NC = 2
_NS = 16
_NW = _NC * _NS
_DPW = _FEAT_DIM // _NW
_SUB = 4096
_NSUB = _BATCH // _SUB
_L = 16


def _center_loss_body(labels_hbm, ft_hbm, ct_hbm, out_hbm,
                      crow_v, lab_v, frow_v, acc_v, csem, fsem, lsem):
    wid = lax.axis_index("s") * _NC + lax.axis_index("c")

    acc = jnp.zeros((_L,), jnp.float32)
    for u in range(_DPW):
        d = wid * _DPW + u
        ccopy = pltpu.async_copy(ct_hbm.at[d, :], crow_v, csem)
        for ch in range(_NSUB):
            lcopy = pltpu.async_copy(
                labels_hbm.at[pl.ds(ch * _SUB, _SUB)], lab_v, lsem)
            fcopy = pltpu.async_copy(
                ft_hbm.at[d, pl.ds(ch * _SUB, _SUB)], frow_v, fsem)
            if ch == 0:
                ccopy.wait()
            lcopy.wait()
            fcopy.wait()

            def grp_body(g, acc):
                lv = lab_v[pl.ds(g * _L, _L)]
                cv = plsc.load_gather(crow_v, [lv])
                fv = frow_v[pl.ds(g * _L, _L)]
                dd = fv - cv
                return acc + dd * dd

            acc = lax.fori_loop(0, _SUB // _L, grp_body, acc, unroll=8)

    acc_v[...] = acc
    pltpu.sync_copy(acc_v, out_hbm.at[wid])


@jax.jit
def _center_loss_sc(labels, features, centers):
    ct = centers.T
    ft = features.T
    mesh = plsc.VectorSubcoreMesh(core_axis_name="c", subcore_axis_name="s")
    partials = pl.kernel(
        _center_loss_body,
        mesh=mesh,
        compiler_params=pltpu.CompilerParams(needs_layout_passes=False),
        out_type=jax.ShapeDtypeStruct((_NW, _L), jnp.float32),
        scratch_types=[
            pltpu.VMEM((_NUM_CLASSES,), jnp.float32),
            pltpu.VMEM((_SUB,), jnp.int32),
            pltpu.VMEM((_SUB,), jnp.float32),
            pltpu.VMEM((_L,), jnp.float32),
            pltpu.SemaphoreType.DMA,
            pltpu.SemaphoreType.DMA,
            pltpu.SemaphoreType.DMA,
        ],
    )(labels, ft, ct)
    return jnp.sum(partials) * (0.5 / _BATCH)


def kernel(features, labels, centers):
    return _center_loss_sc(labels.astype(jnp.int32), features, centers)


# speedup vs baseline: 3.2375x; 1.1240x over previous
import jax
import jax.numpy as jnp
from jax import lax
from jax.experimental import pallas as pl
from jax.experimental.pallas import tpu as pltpu
from jax.experimental.pallas import tpu_sc as plsc

_NUM_CLASSES = 100000
_FEAT_DIM = 64
_BATCH = 16384

_NC = 2
_NS = 16
_NW = _NC * _NS
_DPW = _FEAT_DIM // _NW
_SUB = 4096
_NSUB = _BATCH // _SUB
_L = 16


def _center_loss_body(labels_hbm, ft_hbm, ct_hbm, out_hbm,
                      crow_v, lab_v, frow_v, acc_v, csem, fsem, lsem):
    wid = lax.axis_index("s") * _NC + lax.axis_index("c")

    lcopy = pltpu.async_copy(labels_hbm.at[...], lab_v, lsem)

    acc = jnp.zeros((_L,), jnp.float32)
    for u in range(_DPW):
        d = wid * _DPW + u
        ccopy = pltpu.async_copy(ct_hbm.at[d, :], crow_v, csem)

        def fcopy(ch):
            return pltpu.async_copy(
                ft_hbm.at[d, pl.ds(ch * _SUB, _SUB)], frow_v.at[ch % 2], fsem)

        pending = [fcopy(0), fcopy(1)]
        if u == 0:
            lcopy.wait()
        ccopy.wait()
        for ch in range(_NSUB):
            pending[ch].wait()

            def grp_body(g, acc):
                lv = lab_v[pl.ds(ch * _SUB + g * _L, _L)]
                cv = plsc.load_gather(crow_v, [lv])
                fv = frow_v[ch % 2, pl.ds(g * _L, _L)]
                dd = fv - cv
                return acc + dd * dd

            acc = lax.fori_loop(0, _SUB // _L, grp_body, acc, unroll=8)
            if ch + 2 < _NSUB:
                pending.append(fcopy(ch + 2))

    acc_v[...] = acc
    pltpu.sync_copy(acc_v, out_hbm.at[wid])


@jax.jit
def _center_loss_sc(labels, features, centers):
    ct = centers.T
    ft = features.T
    mesh = plsc.VectorSubcoreMesh(core_axis_name="c", subcore_axis_name="s")
    partials = pl.kernel(
        _center_loss_body,
        mesh=mesh,
        compiler_params=pltpu.CompilerParams(needs_layout_passes=False),
        out_type=jax.ShapeDtypeStruct((_NW, _L), jnp.float32),
        scratch_types=[
            pltpu.VMEM((_NUM_CLASSES,), jnp.float32),
            pltpu.VMEM((_BATCH,), jnp.int32),
            pltpu.VMEM((2, _SUB), jnp.float32),
            pltpu.VMEM((_L,), jnp.float32),
            pltpu.SemaphoreType.DMA,
            pltpu.SemaphoreType.DMA,
            pltpu.SemaphoreType.DMA,
        ],
    )(labels, ft, ct)
    return jnp.sum(partials) * (0.5 / _BATCH)


def kernel(features, labels, centers):
    return _center_loss_sc(labels.astype(jnp.int32), features, centers)


# speedup vs baseline: 3.2456x; 1.0025x over previous
import jax
import jax.numpy as jnp
from jax import lax
from jax.experimental import pallas as pl
from jax.experimental.pallas import tpu as pltpu
from jax.experimental.pallas import tpu_sc as plsc

_NUM_CLASSES = 100000
_FEAT_DIM = 64
_BATCH = 16384

_NC = 2
_NS = 16
_NW = _NC * _NS
_DPW = _FEAT_DIM // _NW
_SUB = 4096
_NSUB = _BATCH // _SUB
_L = 16


def _center_loss_body(labels_hbm, ft_hbm, ct_hbm, out_hbm,
                      crow_v, lab_v, frow_v, acc_v, csem, fsem, lsem):
    wid = lax.axis_index("s") * _NC + lax.axis_index("c")

    lcopy = pltpu.async_copy(labels_hbm.at[...], lab_v, lsem)

    accs = tuple(jnp.zeros((_L,), jnp.float32) for _ in range(8))
    for u in range(_DPW):
        d = wid * _DPW + u
        ccopy = pltpu.async_copy(ct_hbm.at[d, :], crow_v, csem)

        def fcopy(ch):
            return pltpu.async_copy(
                ft_hbm.at[d, pl.ds(ch * _SUB, _SUB)], frow_v.at[ch % 2], fsem)

        pending = [fcopy(0), fcopy(1)]
        if u == 0:
            lcopy.wait()
        ccopy.wait()
        for ch in range(_NSUB):
            pending[ch].wait()

            def grp_body(g, accs):
                lv = lab_v[pl.ds(ch * _SUB + g * _L, _L)]
                cv = plsc.load_gather(crow_v, [lv])
                fv = frow_v[ch % 2, pl.ds(g * _L, _L)]
                dd = fv - cv
                return accs[1:] + (accs[0] + dd * dd,)

            accs = lax.fori_loop(0, _SUB // _L, grp_body, accs, unroll=8)
            if ch + 2 < _NSUB:
                pending.append(fcopy(ch + 2))

    acc = accs[0]
    for a in accs[1:]:
        acc = acc + a
    acc_v[...] = acc
    pltpu.sync_copy(acc_v, out_hbm.at[wid])


@jax.jit
def _center_loss_sc(labels, features, centers):
    ct = centers.T
    ft = features.T
    mesh = plsc.VectorSubcoreMesh(core_axis_name="c", subcore_axis_name="s")
    partials = pl.kernel(
        _center_loss_body,
        mesh=mesh,
        compiler_params=pltpu.CompilerParams(needs_layout_passes=False),
        out_type=jax.ShapeDtypeStruct((_NW, _L), jnp.float32),
        scratch_types=[
            pltpu.VMEM((_NUM_CLASSES,), jnp.float32),
            pltpu.VMEM((_BATCH,), jnp.int32),
            pltpu.VMEM((2, _SUB), jnp.float32),
            pltpu.VMEM((_L,), jnp.float32),
            pltpu.SemaphoreType.DMA,
            pltpu.SemaphoreType.DMA,
            pltpu.SemaphoreType.DMA,
        ],
    )(labels, ft, ct)
    return jnp.sum(partials) * (0.5 / _BATCH)


def kernel(features, labels, centers):
    return _center_loss_sc(labels.astype(jnp.int32), features, centers)
